# Initial kernel scaffold; baseline (speedup 1.0000x reference)
import jax
import jax.numpy as jnp
from jax.experimental import pallas as pl


def kernel(x, edge, W0, gamma0, beta0, anchor, sigma_p, W1, gamma1, beta1):
    raise NotImplementedError("write your pallas kernel here")


# speedup vs baseline: 4.6915x; 4.6915x over previous
import jax
import jax.numpy as jnp
from jax.experimental import pallas as pl

_KNN = 5
_HIGH = jax.lax.Precision.HIGHEST


def _dot(a, b, dims):
    return jax.lax.dot_general(
        a, b, (dims, ((), ())),
        preferred_element_type=jnp.float32, precision=_HIGH)


def _stage1_body(xn_ref, e_ref, w0_ref, g0_ref, b0_ref, anc_ref, sigp_ref,
                 nodes_ref):
    B, N, C = xn_ref.shape

    hs = []
    ssum = jnp.zeros((1, C), jnp.float32)
    for b in range(B):
        x1 = jax.nn.sigmoid(e_ref[b]) * xn_ref[b]
        h = _dot(x1, w0_ref[...], ((1,), (1,)))
        hs.append(h)
        ssum = ssum + jnp.sum(h, axis=0, keepdims=True)
    mean = ssum / (B * N)
    vsum = jnp.zeros((1, C), jnp.float32)
    for b in range(B):
        d = hs[b] - mean
        vsum = vsum + jnp.sum(d * d, axis=0, keepdims=True)
    var = vsum / (B * N)
    scale = g0_ref[...] / jnp.sqrt(var + 1e-5)
    shift = b0_ref[...] - mean * scale

    sig = jax.nn.sigmoid(sigp_ref[...])
    inv2 = 1.0 / (sig * sig)
    anc = anc_ref[...]
    a1 = anc * inv2
    ones_row = jnp.ones((1, C), jnp.float32)
    c0 = _dot(ones_row, anc * a1, ((1,), (1,)))
    ones_col = jnp.ones((N, 1), jnp.float32)

    for b in range(B):
        hn = jnp.maximum(hs[b] * scale + shift, 0.0)
        t1 = _dot(hn * hn, inv2, ((1,), (1,)))
        t2 = _dot(hn, a1, ((1,), (1,)))
        logits = -0.5 * t1 + t2 - 0.5 * c0
        m = jnp.max(logits, axis=1, keepdims=True)
        e = jnp.exp(logits - m)
        sa = e / jnp.sum(e, axis=1, keepdims=True)
        den = _dot(sa, ones_col, ((0,), (0,)))
        sxh = _dot(sa, hn, ((0,), (0,)))
        nodes = (sxh - anc * den) / sig / (den + 1e-9)
        rn = jnp.sqrt(jnp.sum(nodes * nodes, axis=1, keepdims=True))
        nodes = nodes / jnp.maximum(rn, 1e-12)
        fl = jnp.sqrt(jnp.sum(nodes * nodes, keepdims=True))
        nodes = nodes / jnp.maximum(fl, 1e-12)
        nodes_ref[b] = nodes


def _rank_body(xn_ref, m1_ref, w1a_ref, wd_ref, li_ref, q_ref, pm_ref):
    B, N, C = xn_ref.shape
    K = m1_ref.shape[2]

    for b in range(B):
        xb = xn_ref[b]
        m1 = m1_ref[b]
        q_ref[b] = _dot(xb, wd_ref[...], ((1,), (1,)))
        pm_ref[b] = _dot(w1a_ref[...], m1, ((1,), (0,)))
        xv = _dot(xb, m1, ((1,), (0,)))
        xsq = jnp.sum(xb * xb, axis=1, keepdims=True)
        vsq = jnp.sum(m1 * m1, axis=0, keepdims=True)
        d = jnp.sqrt(jnp.maximum(xsq - 2.0 * xv + vsq, 0.0))
        lane = jax.lax.broadcasted_iota(jnp.int32, (N, K), 1).astype(jnp.float32)
        lane5 = jax.lax.broadcasted_iota(jnp.int32, (N, _KNN), 1).astype(jnp.float32)

        dwork = d
        limat = jnp.zeros((N, _KNN), jnp.float32)
        for r in range(_KNN):
            mn = jnp.min(dwork, axis=1, keepdims=True)
            li = jnp.min(jnp.where(dwork <= mn, lane, float(K)), axis=1,
                         keepdims=True)
            limat = jnp.where(lane5 == float(r), li, limat)
            dwork = jnp.where(lane == li, jnp.inf, dwork)
        li_ref[b] = limat


def _edgeconv_body(xn_ref, q_ref, pm_ref, ids_ref, g1_ref, b1_ref, out_ref):
    B, N, C = xn_ref.shape
    K = pm_ref.shape[2]

    sums, mxs, mns = [], [], []
    s1 = jnp.zeros((1, C), jnp.float32)
    s2 = jnp.zeros((1, C), jnp.float32)
    for b in range(B):
        q = q_ref[b]
        pm = pm_ref[b]
        ids = ids_ref[b]
        lane = jax.lax.broadcasted_iota(jnp.int32, (N, K), 1).astype(jnp.float32)
        lane5 = jax.lax.broadcasted_iota(jnp.int32, (N, _KNN), 1).astype(jnp.float32)

        ssum = jnp.zeros((N, C), jnp.float32)
        smax = jnp.full((N, C), -jnp.inf, jnp.float32)
        smin = jnp.full((N, C), jnp.inf, jnp.float32)
        cnt = jnp.zeros((1, K), jnp.float32)
        for m in range(_KNN):
            col = jnp.sum(jnp.where(lane5 == float(m), ids, 0.0), axis=1,
                          keepdims=True)
            mf = (lane == col).astype(jnp.float32)
            g = _dot(mf, pm, ((1,), (1,)))
            ssum = ssum + g
            smax = jnp.maximum(smax, g)
            smin = jnp.minimum(smin, g)
            cnt = cnt + jnp.sum(mf, axis=0, keepdims=True)
        sums.append(ssum)
        mxs.append(smax)
        mns.append(smin)
        s1 = s1 + jnp.sum(ssum, axis=0, keepdims=True) \
            + _KNN * jnp.sum(q, axis=0, keepdims=True)
        s2 = s2 + _dot(cnt, pm * pm, ((1,), (1,))) \
            + 2.0 * jnp.sum(q * ssum, axis=0, keepdims=True) \
            + _KNN * jnp.sum(q * q, axis=0, keepdims=True)

    count = B * N * _KNN
    mean = s1 / count
    var = s2 / count - mean * mean
    a = g1_ref[...] / jnp.sqrt(var + 1e-5)
    bb = b1_ref[...] - mean * a
    for b in range(B):
        meff = jnp.where(a >= 0.0, mxs[b], mns[b])
        y = jnp.maximum(a * (meff + q_ref[b]) + bb, 0.0)
        out_ref[b] = xn_ref[b] + y


def _run(interpret=False):
    def go(xn, en, w0, g0, b0, anc, sigp, w1a, wd, g1, b1):
        B, N, C = xn.shape
        K = anc.shape[0]
        nodes = pl.pallas_call(
            _stage1_body,
            out_shape=jax.ShapeDtypeStruct((B, K, C), jnp.float32),
            interpret=interpret,
        )(xn, en, w0, g0, b0, anc, sigp)
        m1 = nodes.reshape(B, C, K)

        li, q, pm = pl.pallas_call(
            _rank_body,
            out_shape=(
                jax.ShapeDtypeStruct((B, N, _KNN), jnp.float32),
                jax.ShapeDtypeStruct((B, N, C), jnp.float32),
                jax.ShapeDtypeStruct((B, C, K), jnp.float32),
            ),
            interpret=interpret,
        )(xn, m1, w1a, wd)
        ids = li.transpose(0, 2, 1).reshape(B, N, _KNN)

        outn = pl.pallas_call(
            _edgeconv_body,
            out_shape=jax.ShapeDtypeStruct((B, N, C), jnp.float32),
            interpret=interpret,
        )(xn, q, pm, ids, g1, b1)
        return outn
    return go


def kernel(x, edge, W0, gamma0, beta0, anchor, sigma_p, W1, gamma1, beta1):
    B, C, H, W = x.shape
    N = H * W
    xn = x.reshape(B, C, N).transpose(0, 2, 1)
    en = edge.reshape(B, N, 1)
    w1a = W1[:, :C]
    wd = W1[:, C:] - w1a
    outn = _run()(xn, en, W0, gamma0.reshape(1, C), beta0.reshape(1, C),
                  anchor, sigma_p, w1a, wd,
                  gamma1.reshape(1, C), beta1.reshape(1, C))
    return outn.transpose(0, 2, 1).reshape(B, C, H, W)
